# Initial kernel scaffold; baseline (speedup 1.0000x reference)
import functools

import jax
import jax.numpy as jnp
from jax import lax
from jax.experimental import pallas as pl
from jax.experimental.pallas import tpu as pltpu
from jax.experimental.pallas import tpu_sc as plsc

B = 16384
L = 20
D = 32
NC = 2
NS = 16
NW = NC * NS
PER_W = B // NW
C = 128
NCHUNK = PER_W // C
TOKR = (C * L) // 128
LANES = 16


def _sc_body(title_hbm, tokr_hbm, id_hbm, text_hbm, out_hbm,
             tokidx_v, ididx_v, rows_v, idrows_v, out_v, sem):
    wid = lax.axis_index("s") * NC + lax.axis_index("c")
    iota = lax.iota(jnp.int32, LANES)

    for c in range(NCHUNK):
        base = wid * PER_W + c * C
        trow = wid * (PER_W * L // 128) + c * TOKR

        pltpu.sync_copy(title_hbm.at[pl.ds(base, C)], ididx_v)
        pltpu.sync_copy(tokr_hbm.at[pl.ds(trow, TOKR)], tokidx_v)

        copies = [pltpu.async_copy(id_hbm.at[ididx_v], idrows_v, sem)]
        for r in range(TOKR):
            copies.append(pltpu.async_copy(
                text_hbm.at[tokidx_v.at[r]],
                rows_v.at[pl.ds(r * 128, 128)], sem))
        for cp in copies:
            cp.wait()

        def group(g, carry):
            brow = g * LANES + iota
            fb = brow * L
            frows = []
            masks = []
            for l in range(L):
                fl = fb + l
                tv = plsc.load_gather(tokidx_v, [fl // 128, fl % 128])
                frows.append(fl)
                masks.append(jnp.where(tv != 0, 1.0, 0.0))
            cnt = masks[0]
            for l in range(1, L):
                cnt = cnt + masks[l]
            inv = 1.0 / jnp.maximum(cnt, 1.0)

            def dloop(d, dcarry):
                dcol = jnp.full((LANES,), 0, jnp.int32) + d
                s = jnp.zeros((LANES,), jnp.float32)
                for l in range(L):
                    v = plsc.load_gather(rows_v, [frows[l], dcol])
                    s = s + v * masks[l]
                plsc.store_scatter(out_v, [brow, dcol + D], s * inv)
                idv = plsc.load_gather(idrows_v, [brow, dcol])
                plsc.store_scatter(out_v, [brow, dcol], idv)
                return dcarry

            lax.fori_loop(0, D, dloop, 0)
            return carry

        lax.fori_loop(0, C // LANES, group, 0)
        pltpu.sync_copy(out_v, out_hbm.at[pl.ds(base, C)])


_mesh = plsc.VectorSubcoreMesh(core_axis_name="c", subcore_axis_name="s")

_sc_call = functools.partial(
    pl.kernel,
    mesh=_mesh,
    out_type=jax.ShapeDtypeStruct((B, 2 * D), jnp.float32),
    scratch_types=[
        pltpu.VMEM((TOKR, 128), jnp.int32),
        pltpu.VMEM((C,), jnp.int32),
        pltpu.VMEM((C * L, D), jnp.float32),
        pltpu.VMEM((C, D), jnp.float32),
        pltpu.VMEM((C, 2 * D), jnp.float32),
        pltpu.SemaphoreType.DMA,
    ],
)(_sc_body)


@jax.jit
def kernel(title_ids, token_ids, id_table, text_table):
    tok_rows = token_ids.reshape(B * L // 128, 128)
    return _sc_call(title_ids, tok_rows, id_table, text_table)


# speedup vs baseline: 5.3861x; 5.3861x over previous
import functools

import jax
import jax.numpy as jnp
from jax import lax
from jax.experimental import pallas as pl
from jax.experimental.pallas import tpu as pltpu
from jax.experimental.pallas import tpu_sc as plsc

B = 16384
L = 20
D = 32
NC = 2
NS = 16
NW = NC * NS
PER_W = B // NW
C = 128
NCHUNK = PER_W // C
TOKR = (C * L) // 128
LANES = 16


def _sc_body(title_hbm, tokr_hbm, id_hbm, text_hbm, out_hbm,
             tokidx_v, ididx_v, rows_v, idrows_v, out_v, sem):
    wid = lax.axis_index("s") * NC + lax.axis_index("c")
    iota = lax.iota(jnp.int32, LANES)

    wrows = PER_W * L // 128
    pltpu.sync_copy(tokr_hbm.at[pl.ds(wid * wrows, wrows)], tokidx_v)

    for c in range(NCHUNK):
        base = wid * PER_W + c * C

        pltpu.sync_copy(title_hbm.at[pl.ds(base, C)], ididx_v)

        copies = [pltpu.async_copy(id_hbm.at[ididx_v], idrows_v, sem)]
        for r in range(TOKR):
            copies.append(pltpu.async_copy(
                text_hbm.at[tokidx_v.at[c * TOKR + r]],
                rows_v.at[pl.ds(r * 128, 128)], sem))
        for cp in copies:
            cp.wait()

        def group(g, carry):
            brow = g * LANES + iota
            fb = brow * L
            frows = []
            masks = []
            for l in range(L):
                fl = fb + l
                fg = fl + c * (C * L)
                tv = plsc.load_gather(tokidx_v, [fg // 128, fg % 128])
                frows.append(fl)
                masks.append(jnp.where(tv != 0, 1.0, 0.0))
            cnt = masks[0]
            for l in range(1, L):
                cnt = cnt + masks[l]
            inv = 1.0 / jnp.maximum(cnt, 1.0)

            def dloop(d, dcarry):
                dcol = jnp.full((LANES,), 0, jnp.int32) + d
                s = jnp.zeros((LANES,), jnp.float32)
                for l in range(L):
                    v = plsc.load_gather(rows_v, [frows[l], dcol])
                    s = s + v * masks[l]
                plsc.store_scatter(out_v, [brow, dcol + D], s * inv)
                idv = plsc.load_gather(idrows_v, [brow, dcol])
                plsc.store_scatter(out_v, [brow, dcol], idv)
                return dcarry

            lax.fori_loop(0, D, dloop, 0)
            return carry

        lax.fori_loop(0, C // LANES, group, 0)
        pltpu.sync_copy(out_v, out_hbm.at[pl.ds(base, C)])


_mesh = plsc.VectorSubcoreMesh(core_axis_name="c", subcore_axis_name="s")

_sc_call = functools.partial(
    pl.kernel,
    mesh=_mesh,
    compiler_params=pltpu.CompilerParams(use_tc_tiling_on_sc=False,
                                         needs_layout_passes=False),
    out_type=jax.ShapeDtypeStruct((B, 2 * D), jnp.float32),
    scratch_types=[
        pltpu.VMEM((PER_W * L // 128, 128), jnp.int32),
        pltpu.VMEM((C,), jnp.int32),
        pltpu.VMEM((C * L, D), jnp.float32),
        pltpu.VMEM((C, D), jnp.float32),
        pltpu.VMEM((C, 2 * D), jnp.float32),
        pltpu.SemaphoreType.DMA,
    ],
)(_sc_body)


@jax.jit
def kernel(title_ids, token_ids, id_table, text_table):
    tok_rows = token_ids.reshape(B * L // 128, 128)
    return _sc_call(title_ids, tok_rows, id_table, text_table)


# speedup vs baseline: 12.8897x; 2.3931x over previous
import functools

import jax
import jax.numpy as jnp
from jax import lax
from jax.experimental import pallas as pl
from jax.experimental.pallas import tpu as pltpu
from jax.experimental.pallas import tpu_sc as plsc

B = 16384
L = 20
D = 32
NC = 2
NS = 16
NW = NC * NS
PER_W = B // NW
C = 128
NCHUNK = PER_W // C
TOKR = (C * L) // 128
LANES = 16


def _sc_body(title_hbm, tokr_hbm, id_hbm, text_hbm, out_hbm,
             tokidx_v, ididx_v, rows_v, idrows_v, out_v, scale_v, row0_v, sem):
    wid = lax.axis_index("s") * NC + lax.axis_index("c")
    iota = lax.iota(jnp.int32, LANES)

    wrows = PER_W * L // 128
    pltpu.sync_copy(tokr_hbm.at[pl.ds(wid * wrows, wrows)], tokidx_v)
    pltpu.sync_copy(text_hbm.at[pl.ds(0, 8)], row0_v)

    for c in range(NCHUNK):
        base = wid * PER_W + c * C

        pltpu.sync_copy(title_hbm.at[pl.ds(base, C)], ididx_v)

        copies = [pltpu.async_copy(id_hbm.at[ididx_v], idrows_v, sem)]
        for r in range(TOKR):
            copies.append(pltpu.async_copy(
                text_hbm.at[tokidx_v.at[c * TOKR + r]],
                rows_v.at[pl.ds(r * 128, 128)], sem))
        for cp in copies:
            cp.wait()

        def zgroup(g, carry):
            brow = g * LANES + iota
            fb = brow * L + (c * C * L)
            z = jnp.zeros((LANES,), jnp.float32)
            for l in range(L):
                fg = fb + l
                tv = plsc.load_gather(tokidx_v, [fg // 128, fg % 128])
                z = z + jnp.where(tv == 0, 1.0, 0.0)
            inv = 1.0 / jnp.maximum(L * 1.0 - z, 1.0)
            b0 = g * LANES
            scale_v[0, pl.ds(b0, LANES)] = inv
            scale_v[1, pl.ds(b0, LANES)] = z * inv
            return carry

        lax.fori_loop(0, C // LANES, zgroup, 0)

        zv = jnp.zeros((LANES,), jnp.int32)
        ov = zv + 1
        r0a = row0_v[0, pl.ds(0, LANES)]
        r0b = row0_v[0, pl.ds(LANES, LANES)]

        def bloop(b, carry):
            fb = b * L
            a0 = rows_v[fb, pl.ds(0, LANES)]
            a1 = rows_v[fb, pl.ds(LANES, LANES)]
            for l in range(1, L):
                a0 = a0 + rows_v[fb + l, pl.ds(0, LANES)]
                a1 = a1 + rows_v[fb + l, pl.ds(LANES, LANES)]
            bcol = zv + b
            inv = plsc.load_gather(scale_v, [zv, bcol])
            zfi = plsc.load_gather(scale_v, [ov, bcol])
            out_v[b, pl.ds(2 * LANES, LANES)] = a0 * inv - zfi * r0a
            out_v[b, pl.ds(3 * LANES, LANES)] = a1 * inv - zfi * r0b
            out_v[b, pl.ds(0, LANES)] = idrows_v[b, pl.ds(0, LANES)]
            out_v[b, pl.ds(LANES, LANES)] = idrows_v[b, pl.ds(LANES, LANES)]
            return carry

        lax.fori_loop(0, C, bloop, 0)
        pltpu.sync_copy(out_v, out_hbm.at[pl.ds(base, C)])


_mesh = plsc.VectorSubcoreMesh(core_axis_name="c", subcore_axis_name="s")

_sc_call = functools.partial(
    pl.kernel,
    mesh=_mesh,
    compiler_params=pltpu.CompilerParams(use_tc_tiling_on_sc=False,
                                         needs_layout_passes=False),
    out_type=jax.ShapeDtypeStruct((B, 2 * D), jnp.float32),
    scratch_types=[
        pltpu.VMEM((PER_W * L // 128, 128), jnp.int32),
        pltpu.VMEM((C,), jnp.int32),
        pltpu.VMEM((C * L, D), jnp.float32),
        pltpu.VMEM((C, D), jnp.float32),
        pltpu.VMEM((C, 2 * D), jnp.float32),
        pltpu.VMEM((2, C), jnp.float32),
        pltpu.VMEM((8, D), jnp.float32),
        pltpu.SemaphoreType.DMA,
    ],
)(_sc_body)


@jax.jit
def kernel(title_ids, token_ids, id_table, text_table):
    tok_rows = token_ids.reshape(B * L // 128, 128)
    return _sc_call(title_ids, tok_rows, id_table, text_table)
